# baseline (speedup 1.0000x reference)
import functools

import jax
import jax.numpy as jnp
from jax import lax
from jax.experimental import pallas as pl
from jax.experimental.pallas import tpu as pltpu
from jax.experimental.pallas import tpu_sc as plsc

N_NODES = 10000
N_EDGES = 320000
D = 128

NC = 2
NS = 16
NW = NC * NS
EPW = N_EDGES // NW
CHUNK = 40
NCHUNKS = EPW // CHUNK
NRING = 4
DEPTH = NRING - 1
N_ACC = N_NODES
DEG_N = N_NODES
ROWS_PER_TILE = N_NODES // NS


def _sc_aggregate(x, src3, dst3):
  mesh = plsc.VectorSubcoreMesh(core_axis_name="c", subcore_axis_name="s")

  @functools.partial(
      pl.kernel,
      out_type=[
          jax.ShapeDtypeStruct((NC, NS, ROWS_PER_TILE, D), jnp.float32),
          jax.ShapeDtypeStruct((NW, DEG_N), jnp.float32),
      ],
      mesh=mesh,
      scratch_types=[
          [pltpu.VMEM((CHUNK,), jnp.int32) for _ in range(NRING)],
          [pltpu.VMEM((CHUNK,), jnp.int32) for _ in range(NRING)],
          [pltpu.VMEM((CHUNK, D), jnp.float32) for _ in range(NRING)],
          pltpu.VMEM((DEG_N,), jnp.float32),
          pltpu.VMEM_SHARED((N_ACC, D), jnp.float32),
          pltpu.SemaphoreType.DMA((NRING,)),
          pltpu.SemaphoreType.DMA((NRING,)),
      ],
      compiler_params=pltpu.CompilerParams(needs_layout_passes=False),
  )
  def agg_kernel(x_hbm, src_hbm, dst_hbm, agg_out, deg_out,
                 sidx, didx, rows, deg_local, acc, gsems, isems):
    c = lax.axis_index("c")
    s = lax.axis_index("s")
    wid = c * NS + s

    def idx_start(kk, slot):
      base = wid * EPW + kk * CHUNK
      pltpu.async_copy(src_hbm.at[pl.ds(base, CHUNK)], sidx[slot], isems.at[slot])
      pltpu.async_copy(dst_hbm.at[pl.ds(base, CHUNK)], didx[slot], isems.at[slot])

    def idx_wait(kk, slot):
      base = wid * EPW + kk * CHUNK
      pltpu.make_async_copy(src_hbm.at[pl.ds(base, CHUNK)], sidx[slot],
                            isems.at[slot]).wait()
      pltpu.make_async_copy(dst_hbm.at[pl.ds(base, CHUNK)], didx[slot],
                            isems.at[slot]).wait()

    def gather_start(slot):
      pltpu.async_copy(x_hbm.at[sidx[slot]], rows[slot], gsems.at[slot])

    def gather_wait(slot):
      pltpu.make_async_copy(x_hbm.at[sidx[slot]], rows[slot], gsems.at[slot]).wait()

    for slot in range(NRING):
      idx_start(slot, slot)

    zeros16 = jnp.zeros((16,), jnp.float32)

    def zero_rows(i, carry):
      for b in range(NRING):
        for g in range(D // 16):
          rows[b][i, pl.ds(g * 16, 16)] = zeros16
      return carry

    lax.fori_loop(0, CHUNK, zero_rows, 0)

    def zero_deg(i, carry):
      deg_local[pl.ds(i * 16, 16)] = zeros16
      return carry

    lax.fori_loop(0, DEG_N // 16, zero_deg, 0)

    nfull = ROWS_PER_TILE // CHUNK
    for j in range(nfull):
      pltpu.sync_copy(rows[0], acc.at[pl.ds(s * ROWS_PER_TILE + j * CHUNK, CHUNK)])
    rem = ROWS_PER_TILE - nfull * CHUNK
    if rem:
      pltpu.sync_copy(rows[0].at[pl.ds(0, rem)],
                      acc.at[pl.ds(s * ROWS_PER_TILE + nfull * CHUNK, rem)])

    plsc.subcore_barrier()

    ones16 = jnp.ones((16,), jnp.float32)

    def count_deg(slot):
      for g in range(CHUNK // 16):
        idx16 = didx[slot][pl.ds(g * 16, 16)]
        plsc.addupdate_scatter(deg_local, [idx16], ones16)

    def body(k, b, static=False):
      def prefetch():
        idx_wait(k + DEPTH, (b + DEPTH) % NRING)
        gather_start((b + DEPTH) % NRING)

      def refill():
        idx_start(k + NRING, b)

      if static:
        prefetch()
      else:
        pl.when(k <= NCHUNKS - 1 - DEPTH)(prefetch)
      gather_wait(b)
      pltpu.sync_copy(rows[b], acc.at[didx[b]], add=True)
      count_deg(b)
      if static:
        refill()
      else:
        pl.when(k <= NCHUNKS - 1 - NRING)(refill)

    NPEEL = NCHUNKS % NRING
    for slot in range(DEPTH):
      idx_wait(slot, slot)
      gather_start(slot)
    for k0 in range(NPEEL):
      body(k0, k0 % NRING, static=True)

    def pipe_body(j, carry):
      for r in range(NRING):
        body(NRING * j + NPEEL + r, (NPEEL + r) % NRING)
      return carry

    lax.fori_loop(0, (NCHUNKS - NPEEL) // NRING, pipe_body, 0)

    pltpu.sync_copy(deg_local, deg_out.at[wid])
    plsc.subcore_barrier()
    pltpu.sync_copy(acc.at[pl.ds(s * ROWS_PER_TILE, ROWS_PER_TILE)],
                    agg_out.at[c, s])

  return agg_kernel(x, src3, dst3)


BLK = 1000


def _tc_deg_reduce(deg_part):
  def red_kernel(deg_ref, out_ref):
    out_ref[...] = jnp.sum(deg_ref[...], axis=0, keepdims=True)

  return pl.pallas_call(
      red_kernel,
      out_shape=jax.ShapeDtypeStruct((1, DEG_N), jnp.float32),
  )(deg_part)


def _tc_mlp(agg_part, deg_col, w, b2):
  grid = N_NODES // BLK

  def mlp_kernel(agg_ref, deg_ref, w_ref, b_ref, out_ref, sum_ref):
    i = pl.program_id(0)
    agg = agg_ref[0] + agg_ref[1]
    deg = deg_ref[...]
    agg = agg / jnp.maximum(deg, 1.0)
    ms = jnp.mean(agg * agg, axis=1, keepdims=True)
    h = agg / (jnp.sqrt(ms) + 1e-8)
    lin = jnp.dot(h, w_ref[...], preferred_element_type=jnp.float32) + b_ref[...]
    out_ref[...] = jnp.maximum(lin, 0.0)

    @pl.when(i == 0)
    def _init():
      sum_ref[0, 0] = 0.0

    sum_ref[0, 0] += jnp.sum(lin)

    @pl.when(i == grid - 1)
    def _finish():
      sum_ref[0, 0] = sum_ref[0, 0] / (N_NODES * D)

  return pl.pallas_call(
      mlp_kernel,
      grid=(grid,),
      in_specs=[
          pl.BlockSpec((NC, BLK, D), lambda i: (0, i, 0)),
          pl.BlockSpec((BLK, 1), lambda i: (i, 0)),
          pl.BlockSpec((D, D), lambda i: (0, 0)),
          pl.BlockSpec((1, D), lambda i: (0, 0)),
      ],
      out_specs=[
          pl.BlockSpec((BLK, D), lambda i: (i, 0)),
          pl.BlockSpec((1, 1), lambda i: (0, 0), memory_space=pltpu.SMEM),
      ],
      out_shape=[
          jax.ShapeDtypeStruct((N_NODES, D), jnp.float32),
          jax.ShapeDtypeStruct((1, 1), jnp.float32),
      ],
  )(agg_part, deg_col, w, b2)


def kernel(x, edge_index, W, b):
  agg_part, deg_part = _sc_aggregate(x, edge_index[0], edge_index[1])
  agg_part = agg_part.reshape(NC, N_NODES, D)
  deg_col = _tc_deg_reduce(deg_part).reshape(DEG_N, 1)[:N_NODES]
  out, sums = _tc_mlp(agg_part, deg_col, W, b.reshape(1, D))
  return out, sums.reshape(())


# speedup vs baseline: 1.6297x; 1.6297x over previous
import functools

import jax
import jax.numpy as jnp
from jax import lax
from jax.experimental import pallas as pl
from jax.experimental.pallas import tpu as pltpu
from jax.experimental.pallas import tpu_sc as plsc

N_NODES = 10000
N_EDGES = 320000
D = 128

NC = 2
NS = 16
NW = NC * NS
EPW = N_EDGES // NW
CHUNK = 80
NCHUNKS = EPW // CHUNK
NRING = 3
IRING = 6
DEPTH = NRING - 1
RDIST = 4
N_ACC = N_NODES
DEG_N = N_NODES
ROWS_PER_TILE = N_NODES // NS


def _sc_aggregate(x, src3, dst3):
  mesh = plsc.VectorSubcoreMesh(core_axis_name="c", subcore_axis_name="s")

  @functools.partial(
      pl.kernel,
      out_type=[
          jax.ShapeDtypeStruct((NC, NS, ROWS_PER_TILE, D), jnp.float32),
          jax.ShapeDtypeStruct((NW, DEG_N), jnp.float32),
      ],
      mesh=mesh,
      scratch_types=[
          [pltpu.VMEM((CHUNK,), jnp.int32) for _ in range(IRING)],
          [pltpu.VMEM((CHUNK,), jnp.int32) for _ in range(IRING)],
          [pltpu.VMEM((CHUNK, D), jnp.float32) for _ in range(NRING)],
          pltpu.VMEM((DEG_N,), jnp.float32),
          pltpu.VMEM_SHARED((N_ACC, D), jnp.float32),
          pltpu.SemaphoreType.DMA((NRING,)),
          pltpu.SemaphoreType.DMA((IRING,)),
          pltpu.SemaphoreType.DMA((NRING,)),
      ],
      compiler_params=pltpu.CompilerParams(needs_layout_passes=False),
  )
  def agg_kernel(x_hbm, src_hbm, dst_hbm, agg_out, deg_out,
                 sidx, didx, rows, deg_local, acc, gsems, isems, ssems):
    c = lax.axis_index("c")
    s = lax.axis_index("s")
    wid = c * NS + s

    def idx_start(kk, slot):
      base = wid * EPW + kk * CHUNK
      pltpu.async_copy(src_hbm.at[pl.ds(base, CHUNK)], sidx[slot], isems.at[slot])
      pltpu.async_copy(dst_hbm.at[pl.ds(base, CHUNK)], didx[slot], isems.at[slot])

    def idx_wait(kk, slot):
      base = wid * EPW + kk * CHUNK
      pltpu.make_async_copy(src_hbm.at[pl.ds(base, CHUNK)], sidx[slot],
                            isems.at[slot]).wait()
      pltpu.make_async_copy(dst_hbm.at[pl.ds(base, CHUNK)], didx[slot],
                            isems.at[slot]).wait()

    def gather_start(rslot, islot):
      pltpu.async_copy(x_hbm.at[sidx[islot]], rows[rslot], gsems.at[rslot])

    def gather_wait(rslot, islot):
      pltpu.make_async_copy(x_hbm.at[sidx[islot]], rows[rslot],
                            gsems.at[rslot]).wait()

    def scatter_start(rslot, islot):
      pltpu.async_copy(rows[rslot], acc.at[didx[islot]], ssems.at[rslot],
                       add=True)

    def scatter_wait(rslot, islot):
      pltpu.make_async_copy(rows[rslot], acc.at[didx[islot]],
                            ssems.at[rslot]).wait()

    for slot in range(RDIST):
      idx_start(slot, slot)

    zeros16 = jnp.zeros((16,), jnp.float32)

    def zero_rows(i, carry):
      for b in range(NRING):
        for g in range(D // 16):
          rows[b][i, pl.ds(g * 16, 16)] = zeros16
      return carry

    lax.fori_loop(0, CHUNK, zero_rows, 0)

    def zero_deg(i, carry):
      deg_local[pl.ds(i * 16, 16)] = zeros16
      return carry

    lax.fori_loop(0, DEG_N // 16, zero_deg, 0)

    nfull = ROWS_PER_TILE // CHUNK
    for j in range(nfull):
      pltpu.sync_copy(rows[0], acc.at[pl.ds(s * ROWS_PER_TILE + j * CHUNK, CHUNK)])
    rem = ROWS_PER_TILE - nfull * CHUNK
    if rem:
      pltpu.sync_copy(rows[0].at[pl.ds(0, rem)],
                      acc.at[pl.ds(s * ROWS_PER_TILE + nfull * CHUNK, rem)])

    plsc.subcore_barrier()

    ones16 = jnp.ones((16,), jnp.float32)

    def count_deg(slot):
      for g in range(CHUNK // 16):
        idx16 = didx[slot][pl.ds(g * 16, 16)]
        plsc.addupdate_scatter(deg_local, [idx16], ones16)

    def body(k, br, bi, static=False):
      if not static or k >= 1:
        scatter_wait((br + DEPTH) % NRING, (bi + IRING - 1) % IRING)

      def prefetch():
        idx_wait(k + DEPTH, (bi + DEPTH) % IRING)
        gather_start((br + DEPTH) % NRING, (bi + DEPTH) % IRING)

      def refill():
        idx_start(k + RDIST, (bi + RDIST) % IRING)

      if static:
        prefetch()
      else:
        pl.when(k <= NCHUNKS - 1 - DEPTH)(prefetch)
      gather_wait(br, bi)
      scatter_start(br, bi)
      count_deg(bi)
      if static:
        refill()
      else:
        pl.when(k <= NCHUNKS - 1 - RDIST)(refill)

    NPEEL = NCHUNKS % IRING
    for slot in range(DEPTH):
      idx_wait(slot, slot)
      gather_start(slot, slot)
    for k0 in range(NPEEL):
      body(k0, k0 % NRING, k0 % IRING, static=True)

    def pipe_body(j, carry):
      for r in range(IRING):
        k = IRING * j + NPEEL + r
        body(k, (NPEEL + r) % NRING, (NPEEL + r) % IRING)
      return carry

    lax.fori_loop(0, (NCHUNKS - NPEEL) // IRING, pipe_body, 0)

    scatter_wait((NCHUNKS - 1) % NRING, (NCHUNKS - 1) % IRING)

    pltpu.sync_copy(deg_local, deg_out.at[wid])
    plsc.subcore_barrier()
    pltpu.sync_copy(acc.at[pl.ds(s * ROWS_PER_TILE, ROWS_PER_TILE)],
                    agg_out.at[c, s])

  return agg_kernel(x, src3, dst3)


BLK = 1000


def _tc_deg_reduce(deg_part):
  def red_kernel(deg_ref, out_ref):
    out_ref[...] = jnp.sum(deg_ref[...], axis=0, keepdims=True)

  return pl.pallas_call(
      red_kernel,
      out_shape=jax.ShapeDtypeStruct((1, DEG_N), jnp.float32),
  )(deg_part)


def _tc_mlp(agg_part, deg_col, w, b2):
  grid = N_NODES // BLK

  def mlp_kernel(agg_ref, deg_ref, w_ref, b_ref, out_ref, sum_ref):
    i = pl.program_id(0)
    agg = agg_ref[0] + agg_ref[1]
    deg = deg_ref[...]
    agg = agg / jnp.maximum(deg, 1.0)
    ms = jnp.mean(agg * agg, axis=1, keepdims=True)
    h = agg / (jnp.sqrt(ms) + 1e-8)
    lin = jnp.dot(h, w_ref[...], preferred_element_type=jnp.float32) + b_ref[...]
    out_ref[...] = jnp.maximum(lin, 0.0)

    @pl.when(i == 0)
    def _init():
      sum_ref[0, 0] = 0.0

    sum_ref[0, 0] += jnp.sum(lin)

    @pl.when(i == grid - 1)
    def _finish():
      sum_ref[0, 0] = sum_ref[0, 0] / (N_NODES * D)

  return pl.pallas_call(
      mlp_kernel,
      grid=(grid,),
      in_specs=[
          pl.BlockSpec((NC, BLK, D), lambda i: (0, i, 0)),
          pl.BlockSpec((BLK, 1), lambda i: (i, 0)),
          pl.BlockSpec((D, D), lambda i: (0, 0)),
          pl.BlockSpec((1, D), lambda i: (0, 0)),
      ],
      out_specs=[
          pl.BlockSpec((BLK, D), lambda i: (i, 0)),
          pl.BlockSpec((1, 1), lambda i: (0, 0), memory_space=pltpu.SMEM),
      ],
      out_shape=[
          jax.ShapeDtypeStruct((N_NODES, D), jnp.float32),
          jax.ShapeDtypeStruct((1, 1), jnp.float32),
      ],
  )(agg_part, deg_col, w, b2)


def kernel(x, edge_index, W, b):
  agg_part, deg_part = _sc_aggregate(x, edge_index[0], edge_index[1])
  agg_part = agg_part.reshape(NC, N_NODES, D)
  deg_col = _tc_deg_reduce(deg_part).reshape(DEG_N, 1)[:N_NODES]
  out, sums = _tc_mlp(agg_part, deg_col, W, b.reshape(1, D))
  return out, sums.reshape(())


# speedup vs baseline: 1.6778x; 1.0295x over previous
import functools

import jax
import jax.numpy as jnp
from jax import lax
from jax.experimental import pallas as pl
from jax.experimental.pallas import tpu as pltpu
from jax.experimental.pallas import tpu_sc as plsc

N_NODES = 10000
N_EDGES = 320000
D = 128

NC = 2
NS = 16
NW = NC * NS
EPW = N_EDGES // NW
CHUNK = 80
NCHUNKS = EPW // CHUNK
NRING = 3
IRING = 6
DEPTH = NRING - 1
RDIST = 4
N_ACC = N_NODES
DEG_N = N_NODES
ROWS_PER_TILE = N_NODES // NS


def _sc_aggregate(x, src3, dst3):
  mesh = plsc.VectorSubcoreMesh(core_axis_name="c", subcore_axis_name="s")

  @functools.partial(
      pl.kernel,
      out_type=[
          jax.ShapeDtypeStruct((NC, NS, ROWS_PER_TILE, D), jnp.float32),
          jax.ShapeDtypeStruct((NW, DEG_N), jnp.float32),
      ],
      mesh=mesh,
      scratch_types=[
          [pltpu.VMEM((CHUNK,), jnp.int32) for _ in range(IRING)],
          [pltpu.VMEM((CHUNK,), jnp.int32) for _ in range(IRING)],
          [pltpu.VMEM((CHUNK, D), jnp.float32) for _ in range(NRING)],
          pltpu.VMEM((DEG_N,), jnp.float32),
          pltpu.VMEM_SHARED((N_ACC, D), jnp.float32),
          pltpu.SemaphoreType.DMA((NRING,)),
          pltpu.SemaphoreType.DMA((IRING,)),
          pltpu.SemaphoreType.DMA((NRING,)),
      ],
      compiler_params=pltpu.CompilerParams(needs_layout_passes=False),
  )
  def agg_kernel(x_hbm, src_hbm, dst_hbm, agg_out, deg_out,
                 sidx, didx, rows, deg_local, acc, gsems, isems, ssems):
    c = lax.axis_index("c")
    s = lax.axis_index("s")
    wid = c * NS + s

    def idx_start(kk, slot):
      base = wid * EPW + kk * CHUNK
      pltpu.async_copy(src_hbm.at[pl.ds(base, CHUNK)], sidx[slot], isems.at[slot])
      pltpu.async_copy(dst_hbm.at[pl.ds(base, CHUNK)], didx[slot], isems.at[slot])

    def idx_wait(kk, slot):
      base = wid * EPW + kk * CHUNK
      pltpu.make_async_copy(src_hbm.at[pl.ds(base, CHUNK)], sidx[slot],
                            isems.at[slot]).wait()
      pltpu.make_async_copy(dst_hbm.at[pl.ds(base, CHUNK)], didx[slot],
                            isems.at[slot]).wait()

    def gather_start(rslot, islot):
      pltpu.async_copy(x_hbm.at[sidx[islot]], rows[rslot], gsems.at[rslot])

    def gather_wait(rslot, islot):
      pltpu.make_async_copy(x_hbm.at[sidx[islot]], rows[rslot],
                            gsems.at[rslot]).wait()

    def scatter_start(rslot, islot):
      pltpu.async_copy(rows[rslot], acc.at[didx[islot]], ssems.at[rslot],
                       add=True)

    def scatter_wait(rslot, islot):
      pltpu.make_async_copy(rows[rslot], acc.at[didx[islot]],
                            ssems.at[rslot]).wait()

    for slot in range(RDIST):
      idx_start(slot, slot)

    zeros16 = jnp.zeros((16,), jnp.float32)

    def zero_rows(i, carry):
      for b in range(NRING):
        for g in range(D // 16):
          rows[b][i, pl.ds(g * 16, 16)] = zeros16
      return carry

    lax.fori_loop(0, CHUNK, zero_rows, 0)

    def zero_deg(i, carry):
      deg_local[pl.ds(i * 16, 16)] = zeros16
      return carry

    lax.fori_loop(0, DEG_N // 16, zero_deg, 0)

    nfull = ROWS_PER_TILE // CHUNK
    for j in range(nfull):
      pltpu.sync_copy(rows[0], acc.at[pl.ds(s * ROWS_PER_TILE + j * CHUNK, CHUNK)])
    rem = ROWS_PER_TILE - nfull * CHUNK
    if rem:
      pltpu.sync_copy(rows[0].at[pl.ds(0, rem)],
                      acc.at[pl.ds(s * ROWS_PER_TILE + nfull * CHUNK, rem)])

    plsc.subcore_barrier()

    ones16 = jnp.ones((16,), jnp.float32)

    def count_deg(slot):
      for g in range(CHUNK // 16):
        idx16 = didx[slot][pl.ds(g * 16, 16)]
        plsc.addupdate_scatter(deg_local, [idx16], ones16)

    def body(k, br, bi, static=False):
      if not static or k >= 1:
        scatter_wait((br + DEPTH) % NRING, (bi + IRING - 1) % IRING)

      def prefetch():
        idx_wait(k + DEPTH, (bi + DEPTH) % IRING)
        gather_start((br + DEPTH) % NRING, (bi + DEPTH) % IRING)

      def refill():
        idx_start(k + RDIST, (bi + RDIST) % IRING)

      if static:
        prefetch()
      else:
        pl.when(k <= NCHUNKS - 1 - DEPTH)(prefetch)
      gather_wait(br, bi)
      scatter_start(br, bi)
      count_deg(bi)
      if static:
        refill()
      else:
        pl.when(k <= NCHUNKS - 1 - RDIST)(refill)

    NPEEL = NCHUNKS % IRING
    for slot in range(DEPTH):
      idx_wait(slot, slot)
      gather_start(slot, slot)
    for k0 in range(NPEEL):
      body(k0, k0 % NRING, k0 % IRING, static=True)

    def pipe_body(j, carry):
      for r in range(IRING):
        k = IRING * j + NPEEL + r
        body(k, (NPEEL + r) % NRING, (NPEEL + r) % IRING)
      return carry

    lax.fori_loop(0, (NCHUNKS - NPEEL) // IRING, pipe_body, 0)

    scatter_wait((NCHUNKS - 1) % NRING, (NCHUNKS - 1) % IRING)

    pltpu.sync_copy(deg_local, deg_out.at[wid])
    plsc.subcore_barrier()
    pltpu.sync_copy(acc.at[pl.ds(s * ROWS_PER_TILE, ROWS_PER_TILE)],
                    agg_out.at[c, s])

  return agg_kernel(x, src3, dst3)


BLK = 1000


def _tc_mlp(agg_part, deg_part, w, b2):
  grid = N_NODES // BLK

  def mlp_kernel(agg_ref, deg_ref, w_ref, b_ref, out_ref, sum_ref, dcol_ref):
    i = pl.program_id(0)

    @pl.when(i == 0)
    def _deg_col():
      ones = jnp.ones((NW, 1), jnp.float32)
      dcol_ref[...] = lax.dot_general(
          deg_ref[...], ones,
          dimension_numbers=(((0,), (0,)), ((), ())),
          preferred_element_type=jnp.float32)

    agg = agg_ref[0] + agg_ref[1]
    deg = dcol_ref[pl.ds(i * BLK, BLK), :]
    agg = agg / jnp.maximum(deg, 1.0)
    ms = jnp.mean(agg * agg, axis=1, keepdims=True)
    h = agg / (jnp.sqrt(ms) + 1e-8)
    lin = jnp.dot(h, w_ref[...], preferred_element_type=jnp.float32) + b_ref[...]
    out_ref[...] = jnp.maximum(lin, 0.0)

    @pl.when(i == 0)
    def _init():
      sum_ref[0, 0] = 0.0

    sum_ref[0, 0] += jnp.sum(lin)

    @pl.when(i == grid - 1)
    def _finish():
      sum_ref[0, 0] = sum_ref[0, 0] / (N_NODES * D)

  return pl.pallas_call(
      mlp_kernel,
      grid=(grid,),
      in_specs=[
          pl.BlockSpec((NC, BLK, D), lambda i: (0, i, 0)),
          pl.BlockSpec((NW, DEG_N), lambda i: (0, 0)),
          pl.BlockSpec((D, D), lambda i: (0, 0)),
          pl.BlockSpec((1, D), lambda i: (0, 0)),
      ],
      out_specs=[
          pl.BlockSpec((BLK, D), lambda i: (i, 0)),
          pl.BlockSpec((1, 1), lambda i: (0, 0), memory_space=pltpu.SMEM),
      ],
      out_shape=[
          jax.ShapeDtypeStruct((N_NODES, D), jnp.float32),
          jax.ShapeDtypeStruct((1, 1), jnp.float32),
      ],
      scratch_shapes=[pltpu.VMEM((DEG_N, 1), jnp.float32)],
  )(agg_part, deg_part, w, b2)


def kernel(x, edge_index, W, b):
  agg_part, deg_part = _sc_aggregate(x, edge_index[0], edge_index[1])
  agg_part = agg_part.reshape(NC, N_NODES, D)
  out, sums = _tc_mlp(agg_part, deg_part, W, b.reshape(1, D))
  return out, sums.reshape(())
